# baseline (speedup 1.0000x reference)
import functools

import jax
import jax.numpy as jnp
from jax import lax
from jax.experimental import pallas as pl
from jax.experimental.pallas import tpu as pltpu
from jax.experimental.pallas import tpu_sc as plsc

VOCAB = 100000
DIM = 64
BATCH = 16384

_NC = 2
_NS = 16
_NW = _NC * _NS
_BPW = BATCH // _NW
_CH = 128
_NCH = _BPW // _CH
_L = 16

_mesh = plsc.VectorSubcoreMesh(core_axis_name="c", subcore_axis_name="s")


@functools.partial(
    pl.kernel,
    mesh=_mesh,
    out_type=jax.ShapeDtypeStruct((BATCH,), jnp.float32),
    compiler_params=pltpu.CompilerParams(
        needs_layout_passes=False, use_tc_tiling_on_sc=False),
    scratch_types=[
        pltpu.VMEM((_NCH, _CH), jnp.int32),
        pltpu.VMEM((_CH, DIM), jnp.float32),
        pltpu.VMEM((_CH, DIM), jnp.float32),
        pltpu.VMEM((_CH, DIM), jnp.float32),
        pltpu.VMEM((_CH, DIM), jnp.float32),
        pltpu.VMEM((_BPW,), jnp.float32),
        pltpu.SemaphoreType.DMA,
        pltpu.SemaphoreType.DMA,
        pltpu.SemaphoreType.DMA,
        pltpu.SemaphoreType.DMA,
    ],
)
def _freq_lookup(table_hbm, idx_hbm, out_hbm, idx_v, r0, r1, r2, r3, out_v,
                 s0, s1, s2, s3):
    wid = lax.axis_index("s") * _NC + lax.axis_index("c")
    pltpu.sync_copy(idx_hbm.at[pl.ds(wid * _NCH, _NCH)], idx_v)
    sems = (s0, s1, s2, s3)
    rows = (r0, r1, r2, r3)
    copies = [
        pltpu.async_copy(table_hbm.at[idx_v.at[j]], rows[j], sems[j])
        for j in range(_NCH)
    ]
    lanes = lax.broadcasted_iota(jnp.int32, (_L,), 0)
    for j in range(_NCH):
        copies[j].wait()

        def group_body(g, _, j=j):
            row = g * _L + lanes
            acc = jnp.zeros((_L,), jnp.float32)
            for d in range(DIM):
                col = (lanes + d) & (DIM - 1)
                acc = acc + plsc.load_gather(rows[j], [row, col])
            out_v[pl.ds(j * _CH + g * _L, _L)] = acc
            return 0

        lax.fori_loop(0, _CH // _L, group_body, 0)
    pltpu.sync_copy(out_v, out_hbm.at[pl.ds(wid * _BPW, _BPW)])


def kernel(train_table, indices):
    idx = indices[:, 1].astype(jnp.int32).reshape(_NW * _NCH, _CH)
    return _freq_lookup(train_table, idx)
